# Initial kernel scaffold; baseline (speedup 1.0000x reference)
import jax
import jax.numpy as jnp
from jax.experimental import pallas as pl


def kernel(x, W1, W2, W3, W4, W5):
    raise NotImplementedError("write your pallas kernel here")


# speedup vs baseline: 10.5263x; 10.5263x over previous
import functools

import jax
import jax.numpy as jnp
from jax import lax
from jax.experimental import pallas as pl
from jax.experimental.pallas import tpu as pltpu
from jax.experimental.pallas import tpu_sc as plsc

K = 5
RC = 128


def _topk_feat_body(N, x_ref, xr_ref, w1_ref, idx_ref, y_ref, z_ref):
    b = pl.program_id(0)
    xb = x_ref[0]
    xr = xr_ref[0]
    d = lax.dot_general(xr, xb, (((0,), (0,)), ((), ())))
    xx_full = jnp.sum(xb * xb, axis=0, keepdims=True)
    xx_chunk = jnp.sum(xr * xr, axis=0, keepdims=True)
    xx_col = lax.transpose(xx_chunk, (1, 0))
    inner = -2.0 * d
    pd = -xx_col - inner - xx_full
    cols = lax.broadcasted_iota(jnp.int32, (RC, N), 1)
    base = b * N
    for j in range(K):
        m = jnp.max(pd, axis=1, keepdims=True)
        am = jnp.min(jnp.where(pd == m, cols, N), axis=1)
        idx_ref[0, j, :] = am + base
        pd = jnp.where(cols == am[:, None], -jnp.inf, pd)
    w1 = w1_ref[...]
    y_ref[0] = lax.dot_general(xr, w1[:, 0:3], (((0,), (1,)), ((), ())))
    z_ref[0] = lax.dot_general(xr, w1[:, 3:6], (((0,), (1,)), ((), ())))


def _topk_feat(x, W1):
    B, C, N = x.shape
    return pl.pallas_call(
        functools.partial(_topk_feat_body, N),
        grid=(B, N // RC),
        in_specs=[
            pl.BlockSpec((1, C, N), lambda b, r: (b, 0, 0)),
            pl.BlockSpec((1, C, RC), lambda b, r: (b, 0, r)),
            pl.BlockSpec((64, 6), lambda b, r: (0, 0)),
        ],
        out_specs=[
            pl.BlockSpec((1, K, RC), lambda b, r: (b, 0, r)),
            pl.BlockSpec((1, RC, 64), lambda b, r: (b, r, 0)),
            pl.BlockSpec((1, RC, 64), lambda b, r: (b, r, 0)),
        ],
        out_shape=[
            jax.ShapeDtypeStruct((B, K, N), jnp.int32),
            jax.ShapeDtypeStruct((B, N, 64), jnp.float32),
            jax.ShapeDtypeStruct((B, N, 64), jnp.float32),
        ],
    )(x, x, W1)


def _gather_sc(Yf, idx3, n_tasks, N):
    info = plsc.get_sparse_core_info()
    nc, ns = info.num_cores, info.num_subcores
    nw = nc * ns
    tpw = -(-n_tasks // nw)
    n_streams = N // 128
    mesh = plsc.VectorSubcoreMesh(core_axis_name="c", subcore_axis_name="s")

    @functools.partial(
        pl.kernel,
        mesh=mesh,
        compiler_params=pltpu.CompilerParams(use_tc_tiling_on_sc=False),
        out_type=jax.ShapeDtypeStruct((n_tasks, N, 64), jnp.float32),
        scratch_types=[
            pltpu.VMEM((n_streams, 128), jnp.int32),
            pltpu.VMEM((N, 64), jnp.float32),
            pltpu.SemaphoreType.DMA,
        ],
    )
    def gk(yf_hbm, idx_hbm, out_hbm, idx_v, rows_v, sem):
        wid = lax.axis_index("s") * nc + lax.axis_index("c")
        for i in range(tpw):
            t = i * nw + wid

            @pl.when(t < n_tasks)
            def _():
                pltpu.sync_copy(idx_hbm.at[t], idx_v)
                cps = [
                    pltpu.async_copy(
                        yf_hbm.at[idx_v.at[jj]],
                        rows_v.at[pl.ds(jj * 128, 128)],
                        sem,
                    )
                    for jj in range(n_streams)
                ]
                for cp in cps:
                    cp.wait()
                pltpu.sync_copy(rows_v, out_hbm.at[t])

    return gk(Yf, idx3)


def _chain_body(g_ref, z_ref, w2_ref, w3_ref, w4_ref, w5_ref, out_ref):
    z = z_ref[0]
    w2, w3, w4, w5 = w2_ref[...], w3_ref[...], w4_ref[...], w5_ref[...]
    x1 = x2 = x3 = x4 = None
    for j in range(K):
        h1 = jax.nn.relu(g_ref[0, j] + z)
        h2 = jax.nn.relu(lax.dot_general(h1, w2, (((1,), (1,)), ((), ()))))
        h3 = jax.nn.relu(lax.dot_general(h2, w3, (((1,), (1,)), ((), ()))))
        h4 = jax.nn.relu(lax.dot_general(h3, w4, (((1,), (1,)), ((), ()))))
        if j == 0:
            x1, x2, x3, x4 = h1, h2, h3, h4
        else:
            x1 = jnp.maximum(x1, h1)
            x2 = jnp.maximum(x2, h2)
            x3 = jnp.maximum(x3, h3)
            x4 = jnp.maximum(x4, h4)
    cat = jnp.concatenate([x1, x2, x3, x4], axis=1)
    out_ref[0] = jax.nn.relu(
        lax.dot_general(w5, cat, (((1,), (1,)), ((), ())))
    )


def _chain(G4, Z, W2, W3, W4, W5):
    B, _, N, _ = G4.shape
    return pl.pallas_call(
        _chain_body,
        grid=(B,),
        in_specs=[
            pl.BlockSpec((1, K, N, 64), lambda b: (b, 0, 0, 0)),
            pl.BlockSpec((1, N, 64), lambda b: (b, 0, 0)),
            pl.BlockSpec((64, 64), lambda b: (0, 0)),
            pl.BlockSpec((128, 64), lambda b: (0, 0)),
            pl.BlockSpec((256, 128), lambda b: (0, 0)),
            pl.BlockSpec((512, 512), lambda b: (0, 0)),
        ],
        out_specs=pl.BlockSpec((1, 512, N), lambda b: (b, 0, 0)),
        out_shape=jax.ShapeDtypeStruct((B, 512, N), jnp.float32),
    )(G4, Z, W2, W3, W4, W5)


def kernel(x, W1, W2, W3, W4, W5):
    B, C, N = x.shape
    idx, Y, Z = _topk_feat(x, W1)
    n_tasks = B * K
    Yf = Y.reshape(B * N, 64)
    idx3 = idx.reshape(n_tasks, N // 128, 128)
    G = _gather_sc(Yf, idx3, n_tasks, N)
    G4 = G.reshape(B, K, N, 64)
    return _chain(G4, Z, W2, W3, W4, W5)


# speedup vs baseline: 13.3055x; 1.2640x over previous
import functools

import jax
import jax.numpy as jnp
from jax import lax
from jax.experimental import pallas as pl
from jax.experimental.pallas import tpu as pltpu
from jax.experimental.pallas import tpu_sc as plsc

K = 5
RC = 128


def _topk_feat_body(N, x_ref, xr_ref, w1_ref, idx_ref, y_ref, z_ref):
    b = pl.program_id(0)
    r = pl.program_id(1)
    xb = x_ref[0]
    xr = xr_ref[0]
    d = lax.dot_general(xr, xb, (((0,), (0,)), ((), ())))
    xx_full = jnp.sum(xb * xb, axis=0, keepdims=True)
    xx_chunk = jnp.sum(xr * xr, axis=0, keepdims=True)
    xx_col = lax.transpose(xx_chunk, (1, 0))
    inner = -2.0 * d
    pd = -xx_col - inner - xx_full
    cols = lax.broadcasted_iota(jnp.int32, (RC, N), 1)
    colsf = cols.astype(jnp.float32)
    base = b * N
    selfmask = cols == (lax.broadcasted_iota(jnp.int32, (RC, 1), 0) + r * RC)
    idx_ref[0, 0, :] = jnp.min(jnp.where(selfmask, cols, N), axis=1) + base
    pd = jnp.where(selfmask, -jnp.inf, pd)
    for j in range(1, K):
        m = jnp.max(pd, axis=1, keepdims=True)
        amf = jnp.min(jnp.where(pd == m, colsf, float(N)), axis=1)
        idx_ref[0, j, :] = amf.astype(jnp.int32) + base
        pd = jnp.where(colsf == amf[:, None], -jnp.inf, pd)
    w1 = w1_ref[...]
    y_ref[0] = lax.dot_general(xr, w1[:, 0:3], (((0,), (1,)), ((), ())))
    z_ref[0] = lax.dot_general(xr, w1[:, 3:6], (((0,), (1,)), ((), ())))


def _topk_feat(x, W1):
    B, C, N = x.shape
    return pl.pallas_call(
        functools.partial(_topk_feat_body, N),
        grid=(B, N // RC),
        in_specs=[
            pl.BlockSpec((1, C, N), lambda b, r: (b, 0, 0)),
            pl.BlockSpec((1, C, RC), lambda b, r: (b, 0, r)),
            pl.BlockSpec((64, 6), lambda b, r: (0, 0)),
        ],
        out_specs=[
            pl.BlockSpec((1, K, RC), lambda b, r: (b, 0, r)),
            pl.BlockSpec((1, RC, 64), lambda b, r: (b, r, 0)),
            pl.BlockSpec((1, RC, 64), lambda b, r: (b, r, 0)),
        ],
        out_shape=[
            jax.ShapeDtypeStruct((B, K, N), jnp.int32),
            jax.ShapeDtypeStruct((B, N, 64), jnp.float32),
            jax.ShapeDtypeStruct((B, N, 64), jnp.float32),
        ],
    )(x, x, W1)


def _gather_sc(Yf, idx3, n_tasks, N):
    info = plsc.get_sparse_core_info()
    nc, ns = info.num_cores, info.num_subcores
    nw = nc * ns
    tpw = -(-n_tasks // nw)
    n_streams = N // 128
    mesh = plsc.VectorSubcoreMesh(core_axis_name="c", subcore_axis_name="s")

    @functools.partial(
        pl.kernel,
        mesh=mesh,
        compiler_params=pltpu.CompilerParams(use_tc_tiling_on_sc=False),
        out_type=jax.ShapeDtypeStruct((n_tasks, N, 64), jnp.float32),
        scratch_types=[
            pltpu.VMEM((n_streams, 128), jnp.int32),
            pltpu.VMEM((N, 64), jnp.float32),
            pltpu.SemaphoreType.DMA,
        ],
    )
    def gk(yf_hbm, idx_hbm, out_hbm, idx_v, rows_v, sem):
        wid = lax.axis_index("s") * nc + lax.axis_index("c")
        for i in range(tpw):
            t = i * nw + wid

            @pl.when(t < n_tasks)
            def _():
                pltpu.sync_copy(idx_hbm.at[t], idx_v)
                cps = [
                    pltpu.async_copy(
                        yf_hbm.at[idx_v.at[jj]],
                        rows_v.at[pl.ds(jj * 128, 128)],
                        sem,
                    )
                    for jj in range(n_streams)
                ]
                for cp in cps:
                    cp.wait()
                pltpu.sync_copy(rows_v, out_hbm.at[t])

    return gk(Yf, idx3)


def _chain_body(g_ref, z_ref, w2_ref, w3_ref, w4_ref, w5_ref, out_ref):
    z = z_ref[0]
    w2, w3, w4, w5 = w2_ref[...], w3_ref[...], w4_ref[...], w5_ref[...]
    x1 = x2 = x3 = x4 = None
    for j in range(K):
        h1 = jax.nn.relu(g_ref[0, j] + z)
        h2 = jax.nn.relu(lax.dot_general(h1, w2, (((1,), (1,)), ((), ()))))
        h3 = jax.nn.relu(lax.dot_general(h2, w3, (((1,), (1,)), ((), ()))))
        h4 = jax.nn.relu(lax.dot_general(h3, w4, (((1,), (1,)), ((), ()))))
        if j == 0:
            x1, x2, x3, x4 = h1, h2, h3, h4
        else:
            x1 = jnp.maximum(x1, h1)
            x2 = jnp.maximum(x2, h2)
            x3 = jnp.maximum(x3, h3)
            x4 = jnp.maximum(x4, h4)
    cat = jnp.concatenate([x1, x2, x3, x4], axis=1)
    out_ref[0] = jax.nn.relu(
        lax.dot_general(w5, cat, (((1,), (1,)), ((), ())))
    )


def _chain(G4, Z, W2, W3, W4, W5):
    B, _, N, _ = G4.shape
    return pl.pallas_call(
        _chain_body,
        grid=(B,),
        in_specs=[
            pl.BlockSpec((1, K, N, 64), lambda b: (b, 0, 0, 0)),
            pl.BlockSpec((1, N, 64), lambda b: (b, 0, 0)),
            pl.BlockSpec((64, 64), lambda b: (0, 0)),
            pl.BlockSpec((128, 64), lambda b: (0, 0)),
            pl.BlockSpec((256, 128), lambda b: (0, 0)),
            pl.BlockSpec((512, 512), lambda b: (0, 0)),
        ],
        out_specs=pl.BlockSpec((1, 512, N), lambda b: (b, 0, 0)),
        out_shape=jax.ShapeDtypeStruct((B, 512, N), jnp.float32),
    )(G4, Z, W2, W3, W4, W5)


def kernel(x, W1, W2, W3, W4, W5):
    B, C, N = x.shape
    idx, Y, Z = _topk_feat(x, W1)
    n_tasks = B * K
    Yf = Y.reshape(B * N, 64)
    idx3 = idx.reshape(n_tasks, N // 128, 128)
    G = _gather_sc(Yf, idx3, n_tasks, N)
    G4 = G.reshape(B, K, N, 64)
    return _chain(G4, Z, W2, W3, W4, W5)
